# Initial kernel scaffold; baseline (speedup 1.0000x reference)
import jax
import jax.numpy as jnp
from jax.experimental import pallas as pl


def kernel(feats, paths, init_feats, path_weight, fc_weight):
    raise NotImplementedError("write your pallas kernel here")


# speedup vs baseline: 3.5615x; 3.5615x over previous
import functools

import jax
import jax.numpy as jnp
from jax import lax
from jax.experimental import pallas as pl
from jax.experimental.pallas import tpu as pltpu
from jax.experimental.pallas import tpu_sc as plsc

_N = 50000
_D = 128
_NUM_PATH = 3
_PATH_LEN = 4
_K = _NUM_PATH * _PATH_LEN
_NW = 32
_RPW = 1568
_N_PAD = _NW * _RPW
_C = 32
_NCH = _RPW // _C
_IDXR_CHUNK = _C * _K // 128
_IDXR_WORKER = _RPW * _K // 128

_mesh = plsc.VectorSubcoreMesh(core_axis_name="c", subcore_axis_name="s")


@functools.partial(
    pl.kernel,
    mesh=_mesh,
    out_type=jax.ShapeDtypeStruct((_N_PAD, _D), jnp.float32),
    scratch_types=[
        pltpu.VMEM((_C * _K,), jnp.int32),
        pltpu.VMEM((_C * _K, _D), jnp.float32),
        pltpu.VMEM((_C, _D), jnp.float32),
        pltpu.VMEM((_PATH_LEN, _D), jnp.float32),
        pltpu.SemaphoreType.DMA,
    ],
)
def _sc_gather_acc(feats_hbm, idx_hbm, pw_hbm, out_hbm,
                   idx_v, rows_v, out_v, pw_v, sem):
    wid = lax.axis_index("s") * 2 + lax.axis_index("c")
    pltpu.sync_copy(pw_hbm, pw_v)

    def chunk_body(ch, carry):
        row0 = wid * _RPW + ch * _C
        pltpu.sync_copy(idx_hbm.at[pl.ds(row0 * _K, _C * _K)], idx_v)
        copies = []
        for g in range(_IDXR_CHUNK):
            copies.append(pltpu.async_copy(
                feats_hbm.at[idx_v.at[pl.ds(g * 128, 128)]],
                rows_v.at[pl.ds(g * 128, 128)],
                sem))
        for cp in copies:
            cp.wait()
        for v in range(_D // 16):
            sl = pl.ds(v * 16, 16)
            pws = tuple(pw_v[j, sl] for j in range(_PATH_LEN))

            def row_body(c, acc_carry, _sl=sl, _pws=pws):
                b = c * _K
                acc = rows_v[b, _sl] * _pws[0]
                for k in range(1, _K):
                    acc = acc + rows_v[b + k, _sl] * _pws[k % _PATH_LEN]
                out_v[c, _sl] = acc
                return acc_carry

            lax.fori_loop(0, _C, row_body, 0)
        pltpu.sync_copy(out_v, out_hbm.at[pl.ds(row0, _C)])
        return carry

    lax.fori_loop(0, _NCH, chunk_body, 0)


_BN = 1024


def _mm_body(x_ref, w_ref, o_ref):
    o_ref[...] = jnp.maximum(
        lax.dot_general(x_ref[...], w_ref[...],
                        (((1,), (1,)), ((), ())),
                        preferred_element_type=jnp.float32),
        0.0)


def _tc_mm_relu(x, w):
    return pl.pallas_call(
        _mm_body,
        grid=(_N_PAD // _BN,),
        in_specs=[
            pl.BlockSpec((_BN, _D), lambda i: (i, 0)),
            pl.BlockSpec((_D, _D), lambda i: (0, 0)),
        ],
        out_specs=pl.BlockSpec((_BN, _D), lambda i: (i, 0)),
        out_shape=jax.ShapeDtypeStruct((_N_PAD, _D), jnp.float32),
    )(x, w)


def kernel(feats, paths, init_feats, path_weight, fc_weight):
    del init_feats
    idx = jnp.transpose(paths, (1, 0, 2)).reshape(_N, _K).astype(jnp.int32)
    idx = jnp.pad(idx, ((0, _N_PAD - _N), (0, 0)))
    idx2 = idx.reshape(-1)
    pw = path_weight[0] * (1.0 / _NUM_PATH)
    acc = _sc_gather_acc(feats, idx2, pw)
    out = _tc_mm_relu(acc, fc_weight)
    return out[:_N]
